# baseline (speedup 1.0000x reference)
import functools

import jax
import jax.numpy as jnp
from jax import lax
from jax.experimental import pallas as pl
from jax.experimental.pallas import tpu as pltpu
from jax.experimental.pallas import tpu_sc as plsc


def kernel(indices, tables):
    B, F = indices.shape
    _, V, D = tables.shape

    info = plsc.get_sparse_core_info()
    NC, NS, L = info.num_cores, info.num_subcores, info.num_lanes
    NW = NC * NS
    total = B * F
    per_w = total // NW
    CH = 128
    n_ch = per_w // CH

    idx2 = indices.reshape(NW, n_ch, CH)
    tab2 = tables.reshape(F * V, D)
    mesh = plsc.VectorSubcoreMesh(core_axis_name="c", subcore_axis_name="s")

    @functools.partial(
        pl.kernel,
        mesh=mesh,
        compiler_params=pltpu.CompilerParams(use_tc_tiling_on_sc=False),
        out_type=jax.ShapeDtypeStruct((total, D), jnp.float32),
        scratch_types=[
            pltpu.VMEM((n_ch, CH), jnp.int32),
            pltpu.VMEM((per_w, D), jnp.float32),
            pltpu.SemaphoreType.DMA,
        ],
    )
    def emb(idx_hbm, tab_hbm, out_hbm, idx_v, rows_v, sem):
        wid = lax.axis_index("s") * NC + lax.axis_index("c")
        pltpu.sync_copy(idx_hbm.at[wid], idx_v)

        lanes = lax.iota(jnp.int32, L)

        def adjust(j, carry):
            base = j * CH
            for l in range(CH // L):
                s = pl.ds(l * L, L)
                f_id = lax.rem(base + l * L + lanes, F)
                idx_v[j, s] = idx_v[j, s] + f_id * V
            return carry

        lax.fori_loop(0, n_ch, adjust, 0)

        copies = []
        for j in range(n_ch):
            copies.append(
                pltpu.async_copy(
                    tab_hbm.at[idx_v.at[j]],
                    rows_v.at[pl.ds(j * CH, CH)],
                    sem,
                )
            )
        for c in copies:
            c.wait()

        pltpu.sync_copy(rows_v, out_hbm.at[pl.ds(wid * per_w, per_w)])

    out = emb(idx2, tab2)
    return out.reshape(B, F, D)


# speedup vs baseline: 1.0214x; 1.0214x over previous
import functools

import jax
import jax.numpy as jnp
from jax import lax
from jax.experimental import pallas as pl
from jax.experimental.pallas import tpu as pltpu
from jax.experimental.pallas import tpu_sc as plsc


def kernel(indices, tables):
    B, F = indices.shape
    _, V, D = tables.shape

    info = plsc.get_sparse_core_info()
    NC, NS, L = info.num_cores, info.num_subcores, info.num_lanes
    NW = NC * NS
    SB = B // NW
    per_w = SB * F
    CH = 128
    n_ch = per_w // CH
    RPL = 128 // D
    OL = per_w * D // 128

    tab_l = tables.reshape(F * V // RPL, 128)
    out_lines = (B * F * D) // 128

    mesh = plsc.VectorSubcoreMesh(core_axis_name="c", subcore_axis_name="s")

    @functools.partial(
        pl.kernel,
        mesh=mesh,
        compiler_params=pltpu.CompilerParams(needs_layout_passes=False),
        out_type=jax.ShapeDtypeStruct((out_lines, 128), jnp.float32),
        scratch_types=[
            pltpu.VMEM((SB, F), jnp.int32),
            pltpu.VMEM((n_ch, CH), jnp.int32),
            pltpu.VMEM((n_ch, CH), jnp.int32),
            pltpu.VMEM((2, CH, 128), jnp.float32),
            pltpu.VMEM((OL, 128), jnp.float32),
            pltpu.SemaphoreType.DMA,
            pltpu.SemaphoreType.DMA,
        ],
    )
    def emb(idx_hbm, tab_hbm, out_hbm, idx_v, line_v, sub_v, lines_v,
            rows_v, sem0, sem1):
        wid = lax.axis_index("s") * NC + lax.axis_index("c")
        pltpu.sync_copy(idx_hbm.at[pl.ds(wid * SB, SB)], idx_v)

        lanes = lax.iota(jnp.int32, L)
        sems = (sem0, sem1)

        def build(j, carry):
            base = j * CH
            for g in range(CH // L):
                lk = base + g * L + lanes
                f = lax.rem(lk, F)
                s = lax.div(lk, F)
                row = plsc.load_gather(idx_v, [s, f]) + f * V
                sl = pl.ds(g * L, L)
                line_v[j, sl] = lax.shift_right_logical(row, 3)
                sub_v[j, sl] = (row & (RPL - 1)) * D
            return carry

        lax.fori_loop(0, n_ch, build, 0, unroll=False)

        def fire(j, buf):
            return pltpu.async_copy(
                tab_hbm.at[line_v.at[j]], lines_v.at[buf], sems[buf])

        c0 = fire(0, 0)
        c1 = fire(1, 1)

        def extract(j, buf):
            for g in range(CH // L):
                lk = j * CH + g * L + lanes
                src_row = g * L + lanes
                sub16 = sub_v[j, pl.ds(g * L, L)]
                dst_row = lax.shift_right_logical(lk, 3)
                dst_col0 = (lk & 7) * D
                bufv = jnp.full((L,), buf, dtype=jnp.int32)
                for c in range(D):
                    vals = plsc.load_gather(
                        lines_v, [bufv, src_row, sub16 + c])
                    plsc.store_scatter(rows_v, [dst_row, dst_col0 + c], vals)

        def step(j, carry):
            for b in range(2):
                jj = j + b
                pltpu.make_async_copy(
                    tab_hbm.at[line_v.at[0]], lines_v.at[b], sems[b]).wait()
                extract(jj, b)

                @pl.when(jj + 2 < n_ch)
                def _():
                    fire(jj + 2, b)
            return carry

        lax.fori_loop(0, n_ch // 2, lambda i, c: step(i * 2, c), 0,
                      unroll=False)

        pltpu.sync_copy(rows_v, out_hbm.at[pl.ds(wid * OL, OL)])

    out = emb(indices, tab_l)
    return out.reshape(B, F, D)


# speedup vs baseline: 8.2094x; 8.0376x over previous
import functools

import jax
import jax.numpy as jnp
from jax import lax
from jax.experimental import pallas as pl
from jax.experimental.pallas import tpu as pltpu
from jax.experimental.pallas import tpu_sc as plsc


def kernel(indices, tables):
    B, F = indices.shape
    _, V, D = tables.shape

    info = plsc.get_sparse_core_info()
    NC, NS, L = info.num_cores, info.num_subcores, info.num_lanes
    NW = NC * NS
    HD = D // 2
    UNITS = F * 2
    VC = 2048
    VSH = 11
    NFULL = V // VC
    TAILV = (V // 128) * 128
    VLAST = TAILV - NFULL * VC
    NCH = NFULL + 2
    NGRP = B // L

    idx_t = jnp.pad(indices.T, ((0, NW - F), (0, 0)))
    tab2 = jnp.swapaxes(tables, 1, 2).reshape(F * D, V)
    tail = jnp.pad(
        jnp.swapaxes(tables[:, TAILV:, :], 1, 2).reshape(F * D, V - TAILV),
        ((0, 0), (0, 128 - (V - TAILV))),
    )

    mesh = plsc.VectorSubcoreMesh(core_axis_name="c", subcore_axis_name="s")

    @functools.partial(
        pl.kernel,
        mesh=mesh,
        compiler_params=pltpu.CompilerParams(needs_layout_passes=False),
        out_type=jax.ShapeDtypeStruct((F, D, B), jnp.float32),
        scratch_types=[
            pltpu.VMEM((8, B), jnp.int32),
            pltpu.VMEM((2, HD, VC), jnp.float32),
            pltpu.VMEM((HD, 128), jnp.float32),
            pltpu.VMEM((HD, B), jnp.float32),
            pltpu.VMEM((B + L,), jnp.int32),
            pltpu.VMEM((B + L,), jnp.int32),
            pltpu.VMEM(((NCH + 1) * L,), jnp.int32),
            pltpu.SMEM((NCH + 1,), jnp.int32),
            pltpu.SemaphoreType.DMA,
            pltpu.SemaphoreType.DMA,
        ],
    )
    def emb(idx_hbm, tab_hbm, tail_hbm, out_hbm, idxblk, slab, tailslab,
            outs, blist, vlist, cur, pref, sem0, sem1):
        wid = lax.axis_index("s") * NC + lax.axis_index("c")
        lanes = lax.iota(jnp.int32, L)
        ones = jnp.ones((L,), jnp.int32)
        zeros16 = jnp.zeros((L,), jnp.int32)

        def unit_body(u):
            f = lax.rem(u, F)
            h = lax.div(u, F)
            row0 = pl.multiple_of(f * D + HD * h, 8)
            rowblk = pl.multiple_of(lax.div(f, 8) * 8, 8)
            r = f - rowblk
            pltpu.sync_copy(idx_hbm.at[pl.ds(rowblk, 8)], idxblk)
            pltpu.sync_copy(tail_hbm.at[pl.ds(row0, HD)], tailslab)

            def zero(i, carry):
                cur[pl.ds(pl.multiple_of(i * L, L), L)] = zeros16
                return carry

            lax.fori_loop(0, NCH + 1, zero, 0, unroll=False)

            def chunk_of(iv):
                return jnp.where(
                    iv >= TAILV, NCH - 1, lax.shift_right_logical(iv, VSH)
                )

            def hist(g, carry):
                iv = idxblk[r, pl.ds(pl.multiple_of(g * L, L), L)]
                c = chunk_of(iv)
                plsc.addupdate_scatter(cur, [c * L + lanes], ones)
                return carry

            lax.fori_loop(0, NGRP, hist, 0, unroll=False)

            def prefix(c, carry):
                pref[c] = carry
                sl = pl.ds(pl.multiple_of(c * L, L), L)
                grp = cur[sl]
                inc = plsc.cumsum(grp)
                cur[sl] = inc - grp + carry
                return carry + jnp.sum(grp)

            total = lax.fori_loop(0, NCH, prefix, 0, unroll=False)
            pref[NCH] = total

            def append(g, carry):
                iv = idxblk[r, pl.ds(pl.multiple_of(g * L, L), L)]
                bv = g * L + lanes
                c = chunk_of(iv)
                addr = c * L + lanes
                base = plsc.load_gather(cur, [addr])
                plsc.store_scatter(blist, [base], bv)
                plsc.store_scatter(vlist, [base], iv)
                plsc.addupdate_scatter(cur, [addr], ones)
                return carry

            lax.fori_loop(0, NGRP, append, 0, unroll=False)

            def fire(c, buf, sem, width):
                return pltpu.async_copy(
                    tab_hbm.at[pl.ds(row0, HD),
                               pl.ds(pl.multiple_of(c * VC, 128), width)],
                    slab.at[buf, :, pl.ds(0, width)],
                    sem,
                )

            def extract_from(src_ref, gather_idx, c):
                s = pref[c]
                e = pref[c + 1]
                n_grp = lax.shift_right_logical(e - s + (L - 1), 4)

                def egroup(k, carry):
                    pos = s + k * L + lanes
                    m = pos < e
                    bv = plsc.load_gather(blist, [pos], mask=m)
                    vv = plsc.load_gather(vlist, [pos], mask=m)
                    for dd in range(HD):
                        ddv = dd + zeros16
                        vals = plsc.load_gather(
                            src_ref, gather_idx(ddv, vv), mask=m)
                        plsc.store_scatter(outs, [ddv, bv], vals, mask=m)
                    return carry

                lax.fori_loop(0, n_grp, egroup, 0, unroll=False)

            def extract(c, buf):
                bufv = buf + zeros16
                extract_from(
                    slab, lambda ddv, vv: [bufv, ddv, vv - c * VC], c)

            fire(0, 0, sem0, VC)

            def pair(cc, carry):
                c0 = cc * 2
                fire(c0 + 1, 1, sem1, VC)
                pltpu.make_async_copy(
                    tab_hbm.at[pl.ds(row0, HD), pl.ds(0, VC)],
                    slab.at[0],
                    sem0,
                ).wait()
                extract(c0, 0)

                @pl.when(cc < NFULL // 2 - 1)
                def _():
                    fire(c0 + 2, 0, sem0, VC)

                @pl.when(cc == NFULL // 2 - 1)
                def _():
                    fire(NFULL, 0, sem0, VLAST)

                pltpu.make_async_copy(
                    tab_hbm.at[pl.ds(row0, HD), pl.ds(0, VC)],
                    slab.at[1],
                    sem1,
                ).wait()
                extract(c0 + 1, 1)
                return carry

            lax.fori_loop(0, NFULL // 2, pair, 0, unroll=False)

            pltpu.make_async_copy(
                tab_hbm.at[pl.ds(row0, HD), pl.ds(0, VLAST)],
                slab.at[0, :, pl.ds(0, VLAST)],
                sem0,
            ).wait()
            extract(NFULL, 0)
            extract_from(
                tailslab, lambda ddv, vv: [ddv, vv - TAILV], NCH - 1)

            pltpu.sync_copy(
                outs, out_hbm.at[f, pl.ds(pl.multiple_of(HD * h, 8), HD)])

        for t in range(2):
            u = wid + NW * t

            @pl.when(u < UNITS)
            def _():
                unit_body(u)

    out_t = emb(idx_t, tab2, tail)
    return out_t.transpose(2, 0, 1)


# speedup vs baseline: 8.9986x; 1.0961x over previous
import functools

import jax
import jax.numpy as jnp
from jax import lax
from jax.experimental import pallas as pl
from jax.experimental.pallas import tpu as pltpu
from jax.experimental.pallas import tpu_sc as plsc


def kernel(indices, tables):
    B, F = indices.shape
    _, V, D = tables.shape

    info = plsc.get_sparse_core_info()
    NC, NS, L = info.num_cores, info.num_subcores, info.num_lanes
    NW = NC * NS
    HD = D // 2
    UNITS = F * 2
    VC = 4096
    VSH = 12
    TAILV = (V // 128) * 128
    NFULL = TAILV // VC
    VLAST = TAILV - NFULL * VC
    NCH = NFULL + 2
    NGRP = B // L

    idx_t = jnp.pad(indices.T, ((0, NW - F), (0, 0))).reshape(
        NW, B // 128, 128)
    tab2 = jnp.swapaxes(tables, 1, 2).reshape(F * D, V)
    tail = jnp.pad(
        jnp.swapaxes(tables[:, TAILV:, :], 1, 2).reshape(F * D, V - TAILV),
        ((0, 0), (0, 128 - (V - TAILV))),
    )

    mesh = plsc.VectorSubcoreMesh(core_axis_name="c", subcore_axis_name="s")

    @functools.partial(
        pl.kernel,
        mesh=mesh,
        compiler_params=pltpu.CompilerParams(needs_layout_passes=False),
        out_type=jax.ShapeDtypeStruct((F, D, B), jnp.float32),
        scratch_types=[
            pltpu.VMEM((B // 128, 128), jnp.int32),
            pltpu.VMEM((2, HD, VC), jnp.float32),
            pltpu.VMEM((HD, 128), jnp.float32),
            pltpu.VMEM((HD, B), jnp.float32),
            pltpu.VMEM((B + L,), jnp.int32),
            pltpu.VMEM((B + L,), jnp.int32),
            pltpu.VMEM(((NCH + 1) * L,), jnp.int32),
            pltpu.SMEM((NCH + 1,), jnp.int32),
            pltpu.SemaphoreType.DMA,
            pltpu.SemaphoreType.DMA,
        ],
    )
    def emb(idx_hbm, tab_hbm, tail_hbm, out_hbm, idxblk, slab, tailslab,
            outs, blist, vlist, cur, pref, sem0, sem1):
        wid = lax.axis_index("s") * NC + lax.axis_index("c")
        lanes = lax.iota(jnp.int32, L)
        ones = jnp.ones((L,), jnp.int32)
        zeros16 = jnp.zeros((L,), jnp.int32)

        def unit_body(u):
            f = lax.rem(u, F)
            h = lax.div(u, F)
            row0 = pl.multiple_of(f * D + HD * h, 8)
            pltpu.sync_copy(idx_hbm.at[f], idxblk)
            pltpu.sync_copy(tail_hbm.at[pl.ds(row0, HD)], tailslab)

            def zero(i, carry):
                cur[pl.ds(pl.multiple_of(i * L, L), L)] = zeros16
                return carry

            lax.fori_loop(0, NCH + 1, zero, 0, unroll=False)

            def chunk_of(iv):
                return jnp.where(
                    iv >= TAILV, NCH - 1, lax.shift_right_logical(iv, VSH)
                )

            def hist(g, carry):
                iv = idxblk[lax.div(g, 8),
                            pl.ds(pl.multiple_of(lax.rem(g, 8) * L, L), L)]
                c = chunk_of(iv)
                plsc.addupdate_scatter(cur, [c * L + lanes], ones)
                return carry

            lax.fori_loop(0, NGRP, hist, 0, unroll=False)

            def prefix(c, carry):
                pref[c] = carry
                sl = pl.ds(pl.multiple_of(c * L, L), L)
                grp = cur[sl]
                inc = plsc.cumsum(grp)
                cur[sl] = inc - grp + carry
                return carry + jnp.sum(grp)

            total = lax.fori_loop(0, NCH, prefix, 0, unroll=False)
            pref[NCH] = total

            def append(g, carry):
                iv = idxblk[lax.div(g, 8),
                            pl.ds(pl.multiple_of(lax.rem(g, 8) * L, L), L)]
                bv = g * L + lanes
                c = chunk_of(iv)
                addr = c * L + lanes
                base = plsc.load_gather(cur, [addr])
                plsc.store_scatter(blist, [base], bv)
                plsc.store_scatter(vlist, [base], iv)
                plsc.addupdate_scatter(cur, [addr], ones)
                return carry

            lax.fori_loop(0, NGRP, append, 0, unroll=False)

            def fire(c, buf, sem, width):
                return pltpu.async_copy(
                    tab_hbm.at[pl.ds(row0, HD),
                               pl.ds(pl.multiple_of(c * VC, 128), width)],
                    slab.at[buf, :, pl.ds(0, width)],
                    sem,
                )

            def extract_from(src_ref, gather_idx, c):
                s = pref[c]
                e = pref[c + 1]
                n_grp = lax.shift_right_logical(e - s + (L - 1), 4)

                def egroup(k, carry):
                    pos = s + k * L + lanes
                    m = pos < e
                    bv = plsc.load_gather(blist, [pos], mask=m)
                    vv = plsc.load_gather(vlist, [pos], mask=m)
                    for dd in range(HD):
                        ddv = dd + zeros16
                        vals = plsc.load_gather(
                            src_ref, gather_idx(ddv, vv), mask=m)
                        plsc.store_scatter(outs, [ddv, bv], vals, mask=m)
                    return carry

                lax.fori_loop(0, n_grp, egroup, 0, unroll=False)

            def extract(c, buf):
                bufv = buf + zeros16
                extract_from(
                    slab, lambda ddv, vv: [bufv, ddv, vv - c * VC], c)

            fire(0, 0, sem0, VC)

            def pair(cc, carry):
                c0 = cc * 2
                fire(c0 + 1, 1, sem1, VC)
                pltpu.make_async_copy(
                    tab_hbm.at[pl.ds(row0, HD), pl.ds(0, VC)],
                    slab.at[0],
                    sem0,
                ).wait()
                extract(c0, 0)

                @pl.when(cc < NFULL // 2 - 1)
                def _():
                    fire(c0 + 2, 0, sem0, VC)

                @pl.when(cc == NFULL // 2 - 1)
                def _():
                    fire(NFULL, 0, sem0, VLAST)

                pltpu.make_async_copy(
                    tab_hbm.at[pl.ds(row0, HD), pl.ds(0, VC)],
                    slab.at[1],
                    sem1,
                ).wait()
                extract(c0 + 1, 1)
                return carry

            lax.fori_loop(0, NFULL // 2, pair, 0, unroll=False)

            pltpu.make_async_copy(
                tab_hbm.at[pl.ds(row0, HD), pl.ds(0, VLAST)],
                slab.at[0, :, pl.ds(0, VLAST)],
                sem0,
            ).wait()
            extract(NFULL, 0)
            extract_from(
                tailslab, lambda ddv, vv: [ddv, vv - TAILV], NCH - 1)

            pltpu.sync_copy(
                outs, out_hbm.at[f, pl.ds(pl.multiple_of(HD * h, 8), HD)])

        for t in range(2):
            u = wid + NW * t

            @pl.when(u < UNITS)
            def _():
                unit_body(u)

    out_t = emb(idx_t, tab2, tail)
    return out_t.transpose(2, 0, 1)


# speedup vs baseline: 9.2324x; 1.0260x over previous
import functools

import jax
import jax.numpy as jnp
from jax import lax
from jax.experimental import pallas as pl
from jax.experimental.pallas import tpu as pltpu
from jax.experimental.pallas import tpu_sc as plsc


def kernel(indices, tables):
    B, F = indices.shape
    _, V, D = tables.shape

    info = plsc.get_sparse_core_info()
    NC, NS, L = info.num_cores, info.num_subcores, info.num_lanes
    NW = NC * NS
    HD = D // 2
    UNITS = F * 2
    VC = 4096
    VSH = 12
    TAILV = (V // 128) * 128
    NFULL = TAILV // VC
    VLAST = TAILV - NFULL * VC
    NCH = NFULL + 2
    NGRP = B // L

    idx_t = jnp.pad(indices.T, ((0, NW - F), (0, 0))).reshape(
        NW, B // 128, 128)
    tab2 = jnp.swapaxes(tables, 1, 2).reshape(F * D, V)
    tail = jnp.pad(
        jnp.swapaxes(tables[:, TAILV:, :], 1, 2).reshape(F * D, V - TAILV),
        ((0, 0), (0, 128 - (V - TAILV))),
    )

    mesh = plsc.VectorSubcoreMesh(core_axis_name="c", subcore_axis_name="s")

    @functools.partial(
        pl.kernel,
        mesh=mesh,
        compiler_params=pltpu.CompilerParams(needs_layout_passes=False),
        out_type=jax.ShapeDtypeStruct((F, D, B), jnp.float32),
        scratch_types=[
            pltpu.VMEM((B // 128, 128), jnp.int32),
            pltpu.VMEM((2, HD, VC), jnp.float32),
            pltpu.VMEM((HD, 128), jnp.float32),
            pltpu.VMEM((HD, B), jnp.float32),
            pltpu.VMEM((B + L,), jnp.int32),
            pltpu.VMEM((B + L,), jnp.int32),
            pltpu.VMEM(((NCH + 1) * L,), jnp.int32),
            pltpu.SMEM((NCH + 1,), jnp.int32),
            pltpu.SemaphoreType.DMA,
            pltpu.SemaphoreType.DMA,
        ],
    )
    def emb(idx_hbm, tab_hbm, tail_hbm, out_hbm, idxblk, slab, tailslab,
            outs, blist, vlist, cur, pref, sem0, sem1):
        wid = lax.axis_index("s") * NC + lax.axis_index("c")
        lanes = lax.iota(jnp.int32, L)
        ones = jnp.ones((L,), jnp.int32)
        zeros16 = jnp.zeros((L,), jnp.int32)

        def unit_body(u):
            f = lax.rem(u, F)
            h = lax.div(u, F)
            row0 = pl.multiple_of(f * D + HD * h, 8)

            def fire(c, buf, sem, width):
                return pltpu.async_copy(
                    tab_hbm.at[pl.ds(row0, HD),
                               pl.ds(pl.multiple_of(c * VC, 128), width)],
                    slab.at[buf, :, pl.ds(0, width)],
                    sem,
                )

            fire(0, 0, sem0, VC)
            fire(1, 1, sem1, VC)
            pltpu.sync_copy(idx_hbm.at[f], idxblk)
            pltpu.sync_copy(tail_hbm.at[pl.ds(row0, HD)], tailslab)

            def zero(i, carry):
                cur[pl.ds(pl.multiple_of(i * L, L), L)] = zeros16
                return carry

            lax.fori_loop(0, NCH + 1, zero, 0, unroll=False)

            def chunk_of(iv):
                return jnp.where(
                    iv >= TAILV, NCH - 1, lax.shift_right_logical(iv, VSH)
                )

            def hist(g, carry):
                iv = idxblk[lax.div(g, 8),
                            pl.ds(pl.multiple_of(lax.rem(g, 8) * L, L), L)]
                c = chunk_of(iv)
                plsc.addupdate_scatter(cur, [c * L + lanes], ones)
                return carry

            lax.fori_loop(0, NGRP, hist, 0, unroll=False)

            def prefix(c, carry):
                pref[c] = carry
                sl = pl.ds(pl.multiple_of(c * L, L), L)
                grp = cur[sl]
                inc = plsc.cumsum(grp)
                cur[sl] = inc - grp + carry
                return carry + jnp.sum(grp)

            total = lax.fori_loop(0, NCH, prefix, 0, unroll=False)
            pref[NCH] = total

            def append(g, carry):
                iv = idxblk[lax.div(g, 8),
                            pl.ds(pl.multiple_of(lax.rem(g, 8) * L, L), L)]
                bv = g * L + lanes
                c = chunk_of(iv)
                addr = c * L + lanes
                base = plsc.load_gather(cur, [addr])
                plsc.store_scatter(blist, [base], bv)
                plsc.store_scatter(vlist, [base], iv)
                plsc.addupdate_scatter(cur, [addr], ones)
                return carry

            lax.fori_loop(0, NGRP, append, 0, unroll=False)

            def extract_from(src_ref, gather_idx, c):
                s = pref[c]
                e = pref[c + 1]
                n_grp = lax.shift_right_logical(e - s + (L - 1), 4)

                def egroup(k, carry):
                    pos = s + k * L + lanes
                    m = pos < e
                    bv = plsc.load_gather(blist, [pos], mask=m)
                    vv = plsc.load_gather(vlist, [pos], mask=m)
                    for dd in range(HD):
                        ddv = dd + zeros16
                        vals = plsc.load_gather(
                            src_ref, gather_idx(ddv, vv), mask=m)
                        plsc.store_scatter(outs, [ddv, bv], vals, mask=m)
                    return carry

                lax.fori_loop(0, n_grp, egroup, 0, unroll=False)

            def extract(c, buf):
                bufv = buf + zeros16
                extract_from(
                    slab, lambda ddv, vv: [bufv, ddv, vv - c * VC], c)

            def pair(cc, carry):
                c0 = cc * 2
                pltpu.make_async_copy(
                    tab_hbm.at[pl.ds(row0, HD), pl.ds(0, VC)],
                    slab.at[0],
                    sem0,
                ).wait()
                extract(c0, 0)

                @pl.when(c0 + 2 < NFULL)
                def _():
                    fire(c0 + 2, 0, sem0, VC)

                @pl.when(c0 + 2 == NFULL)
                def _():
                    fire(NFULL, 0, sem0, VLAST)

                pltpu.make_async_copy(
                    tab_hbm.at[pl.ds(row0, HD), pl.ds(0, VC)],
                    slab.at[1],
                    sem1,
                ).wait()
                extract(c0 + 1, 1)

                @pl.when(c0 + 3 < NFULL)
                def _():
                    fire(c0 + 3, 1, sem1, VC)

                return carry

            lax.fori_loop(0, NFULL // 2, pair, 0, unroll=False)

            pltpu.make_async_copy(
                tab_hbm.at[pl.ds(row0, HD), pl.ds(0, VLAST)],
                slab.at[0, :, pl.ds(0, VLAST)],
                sem0,
            ).wait()
            extract(NFULL, 0)
            extract_from(
                tailslab, lambda ddv, vv: [ddv, vv - TAILV], NCH - 1)

            pltpu.sync_copy(
                outs, out_hbm.at[f, pl.ds(pl.multiple_of(HD * h, 8), HD)])

        for t in range(2):
            u = wid + NW * t

            @pl.when(u < UNITS)
            def _():
                unit_body(u)

    out_t = emb(idx_t, tab2, tail)
    return out_t.transpose(2, 0, 1)


# speedup vs baseline: 9.9806x; 1.0810x over previous
import functools

import jax
import jax.numpy as jnp
from jax import lax
from jax.experimental import pallas as pl
from jax.experimental.pallas import tpu as pltpu
from jax.experimental.pallas import tpu_sc as plsc


def kernel(indices, tables):
    B, F = indices.shape
    _, V, D = tables.shape

    info = plsc.get_sparse_core_info()
    NC, NS, L = info.num_cores, info.num_subcores, info.num_lanes
    NW = NC * NS
    HD = D // 2
    UNITS = F * 2
    VC = 2048
    VSH = 11
    TAILV = (V // 128) * 128
    NFULL = TAILV // VC
    VLAST = TAILV - NFULL * VC
    NCH = NFULL + 2
    NBUF = 4
    NGRP = B // L

    idx_t = jnp.pad(indices.T, ((0, NW - F), (0, 0))).reshape(
        NW, B // 128, 128)
    tab2 = jnp.swapaxes(tables, 1, 2).reshape(F * D, V)
    tail = jnp.pad(
        jnp.swapaxes(tables[:, TAILV:, :], 1, 2).reshape(F * D, V - TAILV),
        ((0, 0), (0, 128 - (V - TAILV))),
    )

    mesh = plsc.VectorSubcoreMesh(core_axis_name="c", subcore_axis_name="s")

    @functools.partial(
        pl.kernel,
        mesh=mesh,
        compiler_params=pltpu.CompilerParams(needs_layout_passes=False),
        out_type=jax.ShapeDtypeStruct((F, D, B), jnp.float32),
        scratch_types=[
            pltpu.VMEM((B // 128, 128), jnp.int32),
            pltpu.VMEM((4, HD, VC), jnp.float32),
            pltpu.VMEM((HD, 128), jnp.float32),
            pltpu.VMEM((HD, B), jnp.float32),
            pltpu.VMEM((B + L,), jnp.int32),
            pltpu.VMEM((B + L,), jnp.int32),
            pltpu.VMEM(((NCH + 1) * L,), jnp.int32),
            pltpu.SMEM((NCH + 1,), jnp.int32),
            pltpu.SemaphoreType.DMA,
            pltpu.SemaphoreType.DMA,
            pltpu.SemaphoreType.DMA,
            pltpu.SemaphoreType.DMA,
        ],
    )
    def emb(idx_hbm, tab_hbm, tail_hbm, out_hbm, idxblk, slab, tailslab,
            outs, blist, vlist, cur, pref, sem0, sem1, sem2, sem3):
        sems = (sem0, sem1, sem2, sem3)
        wid = lax.axis_index("s") * NC + lax.axis_index("c")
        lanes = lax.iota(jnp.int32, L)
        ones = jnp.ones((L,), jnp.int32)
        zeros16 = jnp.zeros((L,), jnp.int32)

        def unit_body(u):
            f = lax.rem(u, F)
            h = lax.div(u, F)
            row0 = pl.multiple_of(f * D + HD * h, 8)

            def fire(c, buf, sem, width):
                return pltpu.async_copy(
                    tab_hbm.at[pl.ds(row0, HD),
                               pl.ds(pl.multiple_of(c * VC, 128), width)],
                    slab.at[buf, :, pl.ds(0, width)],
                    sem,
                )

            for b in range(NBUF):
                fire(b, b, sems[b], VC)
            pltpu.sync_copy(idx_hbm.at[f], idxblk)
            pltpu.sync_copy(tail_hbm.at[pl.ds(row0, HD)], tailslab)

            def zero(i, carry):
                cur[pl.ds(pl.multiple_of(i * L, L), L)] = zeros16
                return carry

            lax.fori_loop(0, NCH + 1, zero, 0, unroll=False)

            def chunk_of(iv):
                return jnp.where(
                    iv >= TAILV, NCH - 1, lax.shift_right_logical(iv, VSH)
                )

            def hist(g, carry):
                iv = idxblk[lax.div(g, 8),
                            pl.ds(pl.multiple_of(lax.rem(g, 8) * L, L), L)]
                c = chunk_of(iv)
                plsc.addupdate_scatter(cur, [c * L + lanes], ones)
                return carry

            lax.fori_loop(0, NGRP, hist, 0, unroll=False)

            def prefix(c, carry):
                pref[c] = carry
                sl = pl.ds(pl.multiple_of(c * L, L), L)
                grp = cur[sl]
                inc = plsc.cumsum(grp)
                cur[sl] = inc - grp + carry
                return carry + jnp.sum(grp)

            total = lax.fori_loop(0, NCH, prefix, 0, unroll=False)
            pref[NCH] = total

            def append(g, carry):
                iv = idxblk[lax.div(g, 8),
                            pl.ds(pl.multiple_of(lax.rem(g, 8) * L, L), L)]
                bv = g * L + lanes
                c = chunk_of(iv)
                addr = c * L + lanes
                base = plsc.load_gather(cur, [addr])
                plsc.store_scatter(blist, [base], bv)
                plsc.store_scatter(vlist, [base], iv)
                plsc.addupdate_scatter(cur, [addr], ones)
                return carry

            lax.fori_loop(0, NGRP, append, 0, unroll=False)

            def extract_from(src_ref, gather_idx, c):
                s = pref[c]
                e = pref[c + 1]
                n_grp = lax.shift_right_logical(e - s + (L - 1), 4)

                def egroup(k, carry):
                    pos = s + k * L + lanes
                    m = pos < e
                    bv = plsc.load_gather(blist, [pos], mask=m)
                    vv = plsc.load_gather(vlist, [pos], mask=m)
                    for dd in range(HD):
                        ddv = dd + zeros16
                        vals = plsc.load_gather(
                            src_ref, gather_idx(ddv, vv), mask=m)
                        plsc.store_scatter(outs, [ddv, bv], vals, mask=m)
                    return carry

                lax.fori_loop(0, n_grp, egroup, 0, unroll=False)

            def extract(c, buf):
                bufv = buf + zeros16
                extract_from(
                    slab, lambda ddv, vv: [bufv, ddv, vv - c * VC], c)

            def ring(qq, carry):
                for b in range(NBUF):
                    c = qq * NBUF + b
                    pltpu.make_async_copy(
                        tab_hbm.at[pl.ds(row0, HD), pl.ds(0, VC)],
                        slab.at[b],
                        sems[b],
                    ).wait()
                    extract(c, b)

                    @pl.when(c + NBUF < NFULL)
                    def _():
                        fire(c + NBUF, b, sems[b], VC)

                    @pl.when(c + NBUF == NFULL)
                    def _():
                        fire(NFULL, b, sems[b], VLAST)

                return carry

            lax.fori_loop(0, NFULL // NBUF, ring, 0, unroll=False)

            pltpu.make_async_copy(
                tab_hbm.at[pl.ds(row0, HD), pl.ds(0, VLAST)],
                slab.at[0, :, pl.ds(0, VLAST)],
                sems[NFULL % NBUF],
            ).wait()
            extract(NFULL, 0)
            extract_from(
                tailslab, lambda ddv, vv: [ddv, vv - TAILV], NCH - 1)

            pltpu.sync_copy(
                outs, out_hbm.at[f, pl.ds(pl.multiple_of(HD * h, 8), HD)])

        for t in range(2):
            u = wid + NW * t

            @pl.when(u < UNITS)
            def _():
                unit_body(u)

    out_t = emb(idx_t, tab2, tail)
    return out_t.transpose(2, 0, 1)
